# baseline (speedup 1.0000x reference)
import functools

import numpy as np
import jax
import jax.numpy as jnp
from jax import lax
from jax.experimental import pallas as pl
from jax.experimental.pallas import tpu as pltpu


def _tf2x32_np(k1, k2, x0, x1):
    k1 = np.uint32(k1)
    k2 = np.uint32(k2)
    ks = [k1, k2, np.uint32(k1 ^ k2 ^ np.uint32(0x1BD11BDA))]
    rot = (np.array([13, 15, 26, 6], np.uint32),
           np.array([17, 29, 16, 24], np.uint32))
    x = [x0.astype(np.uint32) + ks[0], x1.astype(np.uint32) + ks[1]]
    for i in range(5):
        for r in rot[i % 2]:
            a = x[0] + x[1]
            b = (x[1] << r) | (x[1] >> np.uint32(32 - r))
            x = [a, b ^ a]
        x = [x[0] + ks[(i + 1) % 3],
             x[1] + ks[(i + 2) % 3] + np.uint32(i + 1)]
    return x[0], x[1]


def _step_keys(seed, nsteps):
    key = (np.uint32(np.uint64(seed) >> np.uint64(32)),
           np.uint32(np.uint64(seed) & np.uint64(0xFFFFFFFF)))
    out = []
    for _ in range(nsteps):
        b1, b2 = _tf2x32_np(key[0], key[1],
                            np.array([0, 0], np.uint32),
                            np.array([0, 1], np.uint32))
        key = (b1[0], b2[0])
        out.append((b1[1], b2[1]))
    return out


def _rotl(x, r):
    return lax.shift_left(x, np.uint32(r)) | lax.shift_right_logical(
        x, np.uint32(32 - r))


def _threefry_prekeyed(k1, k2, x1):
    ks2 = k1 ^ k2 ^ np.uint32(0x1BD11BDA)
    ks = (k1, k2, ks2)
    rot = ((13, 15, 26, 6), (17, 29, 16, 24))
    x0 = k1
    for i in range(5):
        for r in rot[i % 2]:
            x0 = x0 + x1
            x1 = _rotl(x1, r)
            x1 = x1 ^ x0
        x0 = x0 + ks[(i + 1) % 3]
        x1 = x1 + (ks[(i + 2) % 3] + np.uint32(i + 1))
    return x0, x1


def _sweeps(l_ref, m_ref, e_sc, k1, k2, bV, *, blk, V, C):
    NC = V // C
    T0 = NC * C
    TW = V - T0
    colc = lax.broadcasted_iota(jnp.int32, (blk, C), 1)
    NEG_INF = jnp.float32(-jnp.inf)
    BIG = jnp.int32(2**31 - 1)

    has_tail = TW > 0
    l_tail = l_ref[:, T0:] if has_tail else None
    m_tail = m_ref[:, T0:] if has_tail else None

    def a_body(c, acc):
        return jnp.maximum(acc, l_ref[:, pl.ds(c * C, C)])
    acc_m = lax.fori_loop(0, NC, a_body, jnp.full((blk, C), NEG_INF), unroll=16)
    m = jnp.max(acc_m, axis=-1, keepdims=True)
    if has_tail:
        m = jnp.maximum(m, jnp.max(l_tail, axis=-1, keepdims=True))

    def b_body(c, acc):
        sl = pl.ds(c * C, C)
        e = jnp.exp(l_ref[:, sl] - m)
        e_sc[:, sl] = e
        return acc + e
    acc_e = lax.fori_loop(0, NC, b_body, jnp.zeros((blk, C), jnp.float32), unroll=16)
    s = jnp.sum(acc_e, axis=-1, keepdims=True)
    if has_tail:
        e_t = jnp.exp(l_tail - m)
        e_sc[:, T0:] = e_t
        s = s + jnp.sum(e_t, axis=-1, keepdims=True)

    def c_body(c, acc):
        sl = pl.ds(c * C, C)
        p = e_sc[:, sl] / s
        e_sc[:, sl] = p
        pm = jnp.where(m_ref[:, sl] > jnp.float32(0.1), p, jnp.float32(0.0))
        return acc + pm
    acc_p = lax.fori_loop(0, NC, c_body, jnp.zeros((blk, C), jnp.float32), unroll=16)
    s2 = jnp.sum(acc_p, axis=-1, keepdims=True)
    if has_tail:
        p_t = e_sc[:, T0:] / s
        e_sc[:, T0:] = p_t
        pm_t = jnp.where(m_tail > jnp.float32(0.1), p_t, jnp.float32(0.0))
        s2 = s2 + jnp.sum(pm_t, axis=-1, keepdims=True)

    def dist_z(p, mc, x1pre):
        d = jnp.where(mc > jnp.float32(0.1), p, jnp.float32(0.0)) / s2
        y0, y1 = _threefry_prekeyed(k1, k2, x1pre)
        bits = y0 ^ y1
        fb = lax.shift_right_logical(bits, np.uint32(9)) | np.uint32(0x3F800000)
        u = lax.bitcast_convert_type(fb, jnp.float32) - jnp.float32(1.0)
        u = jnp.maximum(jnp.float32(np.finfo(np.float32).tiny), u)
        g = -jnp.log(-jnp.log(u))
        z = jnp.log(d + jnp.float32(1e-20)) + g
        return d, z

    colc_u = colc.astype(jnp.uint32)
    bV_u = bV.astype(jnp.uint32)

    def d_body(c, carry):
        zb, cb, db = carry
        sl = pl.ds(c * C, C)
        x1pre = ((bV_u + np.uint32(C) * c.astype(jnp.uint32)) + k2) + colc_u
        d, z = dist_z(e_sc[:, sl], m_ref[:, sl], x1pre)
        upd = z > zb
        return (jnp.where(upd, z, zb), jnp.where(upd, c, cb),
                jnp.where(upd, d, db))
    zb, cb, db = lax.fori_loop(
        0, NC, d_body,
        (jnp.full((blk, C), NEG_INF), jnp.full((blk, C), BIG),
         jnp.zeros((blk, C), jnp.float32)), unroll=16)
    nb = bV + cb * C + colc
    zmax = jnp.max(zb, axis=-1, keepdims=True)
    nidx = jnp.min(jnp.where(zb == zmax, nb, BIG), axis=-1, keepdims=True)
    dsel = jnp.sum(jnp.where(nb == nidx, db, jnp.float32(0.0)),
                   axis=-1, keepdims=True)

    if has_tail:
        colt = lax.broadcasted_iota(jnp.int32, (blk, TW), 1)
        n_t = (bV + T0) + colt
        x1pre_t = n_t.astype(jnp.uint32) + k2
        d_t, z_t = dist_z(e_sc[:, T0:], m_tail, x1pre_t)
        zmax_t = jnp.max(z_t, axis=-1, keepdims=True)
        nidx_t = jnp.min(jnp.where(z_t == zmax_t, n_t, BIG),
                         axis=-1, keepdims=True)
        dsel_t = jnp.sum(jnp.where(n_t == nidx_t, d_t, jnp.float32(0.0)),
                         axis=-1, keepdims=True)
        use_t = zmax_t > zmax
        nidx = jnp.where(use_t, nidx_t, nidx)
        dsel = jnp.where(use_t, dsel_t, dsel)
    return dsel, nidx - bV


def _body_dma(l_hbm, m_hbm, d0_out, d1_out, a0_out, a1_out, l_sc, m_sc,
              e_sc, sem, *, blk, V, keys, C=512):
    i = pl.program_id(0)
    ng = pl.num_programs(0)
    H = blk // 2

    def copies(slot, ib):
        b0 = ib * H
        return [
            pltpu.make_async_copy(l_hbm.at[pl.ds(b0, H), 0, :],
                                  l_sc.at[slot, pl.ds(0, H), :],
                                  sem.at[slot, 0]),
            pltpu.make_async_copy(l_hbm.at[pl.ds(b0, H), 1, :],
                                  l_sc.at[slot, pl.ds(H, H), :],
                                  sem.at[slot, 1]),
            pltpu.make_async_copy(m_hbm.at[pl.ds(b0, H), 0, :],
                                  m_sc.at[slot, pl.ds(0, H), :],
                                  sem.at[slot, 2]),
            pltpu.make_async_copy(m_hbm.at[pl.ds(b0, H), 1, :],
                                  m_sc.at[slot, pl.ds(H, H), :],
                                  sem.at[slot, 3]),
        ]

    @pl.when(i == 0)
    def _():
        for cp in copies(0, 0):
            cp.start()

    @pl.when(i + 1 < ng)
    def _():
        for cp in copies((i + 1) % 2, i + 1):
            cp.start()

    slot = i % 2
    for cp in copies(slot, i):
        cp.wait()

    lv = l_sc.at[slot]
    mv = m_sc.at[slot]
    row = lax.broadcasted_iota(jnp.int32, (blk, 1), 0)
    t1 = row >= H
    b_idx = i * H + jnp.where(t1, row - H, row)
    (k10, k20), (k11, k21) = keys
    k1 = jnp.where(t1, np.uint32(k11), np.uint32(k10)).astype(jnp.uint32)
    k2 = jnp.where(t1, np.uint32(k21), np.uint32(k20)).astype(jnp.uint32)
    bV = b_idx * V
    dsel, vidx = _sweeps(lv, mv, e_sc, k1, k2, bV, blk=blk, V=V, C=C)
    d0_out[...] = dsel[0:H]
    d1_out[...] = dsel[H:]
    a0_out[...] = vidx[0:H]
    a1_out[...] = vidx[H:]


def _sample_3d(logits, mask_raw, blk, keys, interpret=False):
    B, T, V = logits.shape
    H = blk // 2
    d0, d1, a0, a1 = pl.pallas_call(
        functools.partial(_body_dma, blk=blk, V=V, keys=keys),
        grid=(B // H,),
        in_specs=[pl.BlockSpec(memory_space=pl.ANY),
                  pl.BlockSpec(memory_space=pl.ANY)],
        out_specs=[pl.BlockSpec((H, 1), lambda i: (i, 0)),
                   pl.BlockSpec((H, 1), lambda i: (i, 0)),
                   pl.BlockSpec((H, 1), lambda i: (i, 0)),
                   pl.BlockSpec((H, 1), lambda i: (i, 0))],
        out_shape=[jax.ShapeDtypeStruct((B, 1), jnp.float32),
                   jax.ShapeDtypeStruct((B, 1), jnp.float32),
                   jax.ShapeDtypeStruct((B, 1), jnp.int32),
                   jax.ShapeDtypeStruct((B, 1), jnp.int32)],
        scratch_shapes=[pltpu.VMEM((2, blk, V), jnp.float32),
                        pltpu.VMEM((2, blk, V), jnp.float32),
                        pltpu.VMEM((blk, V), jnp.float32),
                        pltpu.SemaphoreType.DMA((2, 4))],
        interpret=interpret,
    )(logits, mask_raw)
    return d0, d1, a0, a1


def kernel(logits, mask_raw):
    B, T, V = logits.shape
    keys = _step_keys(42, T)
    d0, d1, a0, a1 = _sample_3d(logits, mask_raw, 16, keys)
    fwd = jnp.concatenate([d0, d1], axis=1)
    action = a1[:, 0]
    s_dist = d1
    return fwd, action, s_dist


# speedup vs baseline: 1.0062x; 1.0062x over previous
import functools

import numpy as np
import jax
import jax.numpy as jnp
from jax import lax
from jax.experimental import pallas as pl
from jax.experimental.pallas import tpu as pltpu


def _tf2x32_np(k1, k2, x0, x1):
    k1 = np.uint32(k1)
    k2 = np.uint32(k2)
    ks = [k1, k2, np.uint32(k1 ^ k2 ^ np.uint32(0x1BD11BDA))]
    rot = (np.array([13, 15, 26, 6], np.uint32),
           np.array([17, 29, 16, 24], np.uint32))
    x = [x0.astype(np.uint32) + ks[0], x1.astype(np.uint32) + ks[1]]
    for i in range(5):
        for r in rot[i % 2]:
            a = x[0] + x[1]
            b = (x[1] << r) | (x[1] >> np.uint32(32 - r))
            x = [a, b ^ a]
        x = [x[0] + ks[(i + 1) % 3],
             x[1] + ks[(i + 2) % 3] + np.uint32(i + 1)]
    return x[0], x[1]


def _step_keys(seed, nsteps):
    key = (np.uint32(np.uint64(seed) >> np.uint64(32)),
           np.uint32(np.uint64(seed) & np.uint64(0xFFFFFFFF)))
    out = []
    for _ in range(nsteps):
        b1, b2 = _tf2x32_np(key[0], key[1],
                            np.array([0, 0], np.uint32),
                            np.array([0, 1], np.uint32))
        key = (b1[0], b2[0])
        out.append((b1[1], b2[1]))
    return out


def _rotl(x, r):
    return lax.shift_left(x, np.uint32(r)) | lax.shift_right_logical(
        x, np.uint32(32 - r))


def _threefry_prekeyed(k1, k2, x1):
    ks2 = k1 ^ k2 ^ np.uint32(0x1BD11BDA)
    ks = (k1, k2, ks2)
    rot = ((13, 15, 26, 6), (17, 29, 16, 24))
    x0 = k1
    for i in range(5):
        for r in rot[i % 2]:
            x0 = x0 + x1
            x1 = _rotl(x1, r)
            x1 = x1 ^ x0
        x0 = x0 + ks[(i + 1) % 3]
        x1 = x1 + (ks[(i + 2) % 3] + np.uint32(i + 1))
    return x0, x1


def _sweeps(l_ref, m_ref, e_sc, k1, k2, bV, *, blk, V, C):
    NC = V // C
    T0 = NC * C
    TW = V - T0
    colc = lax.broadcasted_iota(jnp.int32, (blk, C), 1)
    NEG_INF = jnp.float32(-jnp.inf)
    BIG = jnp.int32(2**31 - 1)

    has_tail = TW > 0
    l_tail = l_ref[:, T0:] if has_tail else None
    m_tail = m_ref[:, T0:] if has_tail else None

    def a_body(c, acc):
        return jnp.maximum(acc, l_ref[:, pl.ds(c * C, C)])
    acc_m = lax.fori_loop(0, NC, a_body, jnp.full((blk, C), NEG_INF), unroll=16)
    m = jnp.max(acc_m, axis=-1, keepdims=True)
    if has_tail:
        m = jnp.maximum(m, jnp.max(l_tail, axis=-1, keepdims=True))

    def b_body(c, acc):
        sl = pl.ds(c * C, C)
        e = jnp.exp(l_ref[:, sl] - m)
        e_sc[:, sl] = e
        return acc + e
    acc_e = lax.fori_loop(0, NC, b_body, jnp.zeros((blk, C), jnp.float32), unroll=16)
    s = jnp.sum(acc_e, axis=-1, keepdims=True)
    if has_tail:
        e_t = jnp.exp(l_tail - m)
        e_sc[:, T0:] = e_t
        s = s + jnp.sum(e_t, axis=-1, keepdims=True)

    def c_body(c, acc):
        sl = pl.ds(c * C, C)
        p = e_sc[:, sl] / s
        pm = jnp.where(m_ref[:, sl] > jnp.float32(0.1), p, jnp.float32(0.0))
        e_sc[:, sl] = pm
        return acc + pm
    acc_p = lax.fori_loop(0, NC, c_body, jnp.zeros((blk, C), jnp.float32), unroll=16)
    s2 = jnp.sum(acc_p, axis=-1, keepdims=True)
    if has_tail:
        p_t = e_sc[:, T0:] / s
        pm_t = jnp.where(m_tail > jnp.float32(0.1), p_t, jnp.float32(0.0))
        e_sc[:, T0:] = pm_t
        s2 = s2 + jnp.sum(pm_t, axis=-1, keepdims=True)

    def dist_z(pm, x1pre):
        d = pm / s2
        y0, y1 = _threefry_prekeyed(k1, k2, x1pre)
        bits = y0 ^ y1
        fb = lax.shift_right_logical(bits, np.uint32(9)) | np.uint32(0x3F800000)
        u = lax.bitcast_convert_type(fb, jnp.float32) - jnp.float32(1.0)
        u = jnp.maximum(jnp.float32(np.finfo(np.float32).tiny), u)
        g = -jnp.log(-jnp.log(u))
        z = jnp.log(d + jnp.float32(1e-20)) + g
        return d, z

    colc_u = colc.astype(jnp.uint32)
    bV_u = bV.astype(jnp.uint32)

    def d_body(c, carry):
        zb, cb, db = carry
        sl = pl.ds(c * C, C)
        x1pre = ((bV_u + np.uint32(C) * c.astype(jnp.uint32)) + k2) + colc_u
        d, z = dist_z(e_sc[:, sl], x1pre)
        upd = z > zb
        return (jnp.where(upd, z, zb), jnp.where(upd, c, cb),
                jnp.where(upd, d, db))
    zb, cb, db = lax.fori_loop(
        0, NC, d_body,
        (jnp.full((blk, C), NEG_INF), jnp.full((blk, C), BIG),
         jnp.zeros((blk, C), jnp.float32)), unroll=16)
    nb = bV + cb * C + colc
    zmax = jnp.max(zb, axis=-1, keepdims=True)
    nidx = jnp.min(jnp.where(zb == zmax, nb, BIG), axis=-1, keepdims=True)
    dsel = jnp.sum(jnp.where(nb == nidx, db, jnp.float32(0.0)),
                   axis=-1, keepdims=True)

    if has_tail:
        colt = lax.broadcasted_iota(jnp.int32, (blk, TW), 1)
        n_t = (bV + T0) + colt
        x1pre_t = n_t.astype(jnp.uint32) + k2
        d_t, z_t = dist_z(e_sc[:, T0:], x1pre_t)
        zmax_t = jnp.max(z_t, axis=-1, keepdims=True)
        nidx_t = jnp.min(jnp.where(z_t == zmax_t, n_t, BIG),
                         axis=-1, keepdims=True)
        dsel_t = jnp.sum(jnp.where(n_t == nidx_t, d_t, jnp.float32(0.0)),
                         axis=-1, keepdims=True)
        use_t = zmax_t > zmax
        nidx = jnp.where(use_t, nidx_t, nidx)
        dsel = jnp.where(use_t, dsel_t, dsel)
    return dsel, nidx - bV


def _body_dma(l_hbm, m_hbm, d0_out, d1_out, a0_out, a1_out, l_sc, m_sc,
              e_sc, sem, *, blk, V, keys, C=512):
    i = pl.program_id(0)
    ng = pl.num_programs(0)
    H = blk // 2

    def copies(slot, ib):
        b0 = ib * H
        return [
            pltpu.make_async_copy(l_hbm.at[pl.ds(b0, H), 0, :],
                                  l_sc.at[slot, pl.ds(0, H), :],
                                  sem.at[slot, 0]),
            pltpu.make_async_copy(l_hbm.at[pl.ds(b0, H), 1, :],
                                  l_sc.at[slot, pl.ds(H, H), :],
                                  sem.at[slot, 1]),
            pltpu.make_async_copy(m_hbm.at[pl.ds(b0, H), 0, :],
                                  m_sc.at[slot, pl.ds(0, H), :],
                                  sem.at[slot, 2]),
            pltpu.make_async_copy(m_hbm.at[pl.ds(b0, H), 1, :],
                                  m_sc.at[slot, pl.ds(H, H), :],
                                  sem.at[slot, 3]),
        ]

    @pl.when(i == 0)
    def _():
        for cp in copies(0, 0):
            cp.start()

    @pl.when(i + 1 < ng)
    def _():
        for cp in copies((i + 1) % 2, i + 1):
            cp.start()

    slot = i % 2
    for cp in copies(slot, i):
        cp.wait()

    lv = l_sc.at[slot]
    mv = m_sc.at[slot]
    row = lax.broadcasted_iota(jnp.int32, (blk, 1), 0)
    t1 = row >= H
    b_idx = i * H + jnp.where(t1, row - H, row)
    (k10, k20), (k11, k21) = keys
    k1 = jnp.where(t1, np.uint32(k11), np.uint32(k10)).astype(jnp.uint32)
    k2 = jnp.where(t1, np.uint32(k21), np.uint32(k20)).astype(jnp.uint32)
    bV = b_idx * V
    dsel, vidx = _sweeps(lv, mv, e_sc, k1, k2, bV, blk=blk, V=V, C=C)
    d0_out[...] = dsel[0:H]
    d1_out[...] = dsel[H:]
    a0_out[...] = vidx[0:H]
    a1_out[...] = vidx[H:]


def _sample_3d(logits, mask_raw, blk, keys, interpret=False):
    B, T, V = logits.shape
    H = blk // 2
    d0, d1, a0, a1 = pl.pallas_call(
        functools.partial(_body_dma, blk=blk, V=V, keys=keys),
        grid=(B // H,),
        in_specs=[pl.BlockSpec(memory_space=pl.ANY),
                  pl.BlockSpec(memory_space=pl.ANY)],
        out_specs=[pl.BlockSpec((H, 1), lambda i: (i, 0)),
                   pl.BlockSpec((H, 1), lambda i: (i, 0)),
                   pl.BlockSpec((H, 1), lambda i: (i, 0)),
                   pl.BlockSpec((H, 1), lambda i: (i, 0))],
        out_shape=[jax.ShapeDtypeStruct((B, 1), jnp.float32),
                   jax.ShapeDtypeStruct((B, 1), jnp.float32),
                   jax.ShapeDtypeStruct((B, 1), jnp.int32),
                   jax.ShapeDtypeStruct((B, 1), jnp.int32)],
        scratch_shapes=[pltpu.VMEM((2, blk, V), jnp.float32),
                        pltpu.VMEM((2, blk, V), jnp.float32),
                        pltpu.VMEM((blk, V), jnp.float32),
                        pltpu.SemaphoreType.DMA((2, 4))],
        interpret=interpret,
    )(logits, mask_raw)
    return d0, d1, a0, a1


def kernel(logits, mask_raw):
    B, T, V = logits.shape
    keys = _step_keys(42, T)
    d0, d1, a0, a1 = _sample_3d(logits, mask_raw, 16, keys)
    fwd = jnp.concatenate([d0, d1], axis=1)
    action = a1[:, 0]
    s_dist = d1
    return fwd, action, s_dist
